# baseline (speedup 1.0000x reference)
import functools

import jax
import jax.numpy as jnp
from jax import lax
from jax.experimental import pallas as pl
from jax.experimental.pallas import tpu as pltpu
from jax.experimental.pallas import tpu_sc as plsc


_BM = 400


def _mm_body(sa_ref, sb_ref, x_ref, w_ref, o_ref):
    h = sa_ref.shape[0]
    sxa = jnp.dot(sa_ref[...], x_ref[...], preferred_element_type=jnp.float32)
    o_ref[:h, :] = jnp.dot(sxa, w_ref[...], preferred_element_type=jnp.float32)
    sxb = jnp.dot(sb_ref[...], x_ref[...], preferred_element_type=jnp.float32)
    o_ref[h:, :] = jnp.dot(sxb, w_ref[...], preferred_element_type=jnp.float32)


def _spmm(sup, x, w):
    n, k = sup.shape
    d = w.shape[1]
    hb = _BM // 2
    return pl.pallas_call(
        _mm_body,
        grid=(n // _BM,),
        in_specs=[
            pl.BlockSpec((hb, k), lambda i: (2 * i, 0)),
            pl.BlockSpec((hb, k), lambda i: (2 * i + 1, 0)),
            pl.BlockSpec((k, x.shape[1]), lambda i: (0, 0)),
            pl.BlockSpec(w.shape, lambda i: (0, 0)),
        ],
        out_specs=pl.BlockSpec((_BM, d), lambda i: (i, 0)),
        out_shape=jax.ShapeDtypeStruct((n, d), jnp.float32),
    )(sup, sup, x, w)


_NW = 32
_CHUNK = 80
_CPW = 4


def _make_gather(n, d):
    mesh = plsc.VectorSubcoreMesh(core_axis_name="c", subcore_axis_name="s")
    n_chunks = n // _CHUNK
    per_w = _CPW * _CHUNK
    tail_wid = n_chunks // _CPW
    tail_count = n_chunks % _CPW

    @functools.partial(
        pl.kernel,
        mesh=mesh,
        out_type=jax.ShapeDtypeStruct((n, d), jnp.float32),
        scratch_types=[
            pltpu.VMEM((per_w,), jnp.int32),
            pltpu.VMEM((_CPW, _CHUNK, d), jnp.float32),
            pltpu.SemaphoreType.DMA,
            pltpu.SemaphoreType.DMA,
        ],
    )
    def gk(table_hbm, idx_hbm, out_hbm, idx_v, rows_v, gsem, wsem):
        wid = lax.axis_index("s") * 2 + lax.axis_index("c")
        base = wid * per_w

        @pl.when(wid < tail_wid)
        def _full():
            pltpu.sync_copy(idx_hbm.at[pl.ds(base, per_w)], idx_v)
            gathers = [
                pltpu.async_copy(
                    table_hbm.at[idx_v.at[pl.ds(c * _CHUNK, _CHUNK)]],
                    rows_v.at[c], gsem)
                for c in range(_CPW)
            ]
            writes = []
            for c in range(_CPW):
                gathers[c].wait()
                writes.append(pltpu.async_copy(
                    rows_v.at[c],
                    out_hbm.at[pl.ds(base + c * _CHUNK, _CHUNK)],
                    wsem))
            for wr in writes:
                wr.wait()

        @pl.when(wid == tail_wid)
        def _tail():
            for c in range(tail_count):
                pltpu.sync_copy(
                    idx_hbm.at[pl.ds(base + c * _CHUNK, _CHUNK)],
                    idx_v.at[pl.ds(c * _CHUNK, _CHUNK)])
                pltpu.async_copy(
                    table_hbm.at[idx_v.at[pl.ds(c * _CHUNK, _CHUNK)]],
                    rows_v.at[c], gsem).wait()
                pltpu.sync_copy(
                    rows_v.at[c],
                    out_hbm.at[pl.ds(base + c * _CHUNK, _CHUNK)])

    return gk


def kernel(x, selected_index, support, w):
    n = x.shape[0]
    s = _spmm(support[0], x, w)
    for i in range(1, support.shape[0]):
        s = s + _spmm(support[i], x, w)
    return _make_gather(n, w.shape[1])(s, selected_index.astype(jnp.int32))


# speedup vs baseline: 1.0754x; 1.0754x over previous
import functools

import jax
import jax.numpy as jnp
from jax import lax
from jax.experimental import pallas as pl
from jax.experimental.pallas import tpu as pltpu
from jax.experimental.pallas import tpu_sc as plsc


_BM = 400


def _mm_body(s_ref, x_ref, w_ref, o_ref):
    sx = jnp.dot(s_ref[...], x_ref[...], preferred_element_type=jnp.float32)
    o_ref[...] = jnp.dot(sx, w_ref[...], preferred_element_type=jnp.float32)


def _spmm(sup, x, w):
    n, k = sup.shape
    d = w.shape[1]
    return pl.pallas_call(
        _mm_body,
        grid=(n // _BM,),
        in_specs=[
            pl.BlockSpec((_BM, k), lambda i: (i, 0)),
            pl.BlockSpec((k, x.shape[1]), lambda i: (0, 0)),
            pl.BlockSpec(w.shape, lambda i: (0, 0)),
        ],
        out_specs=pl.BlockSpec((_BM, d), lambda i: (i, 0)),
        out_shape=jax.ShapeDtypeStruct((n, d), jnp.float32),
    )(sup, x, w)


_NW = 32
_CHUNK = 80
_CPW = 4


def _make_gather(n, d):
    mesh = plsc.VectorSubcoreMesh(core_axis_name="c", subcore_axis_name="s")
    n_chunks = n // _CHUNK
    per_w = _CPW * _CHUNK
    tail_wid = n_chunks // _CPW
    tail_count = n_chunks % _CPW

    @functools.partial(
        pl.kernel,
        mesh=mesh,
        out_type=jax.ShapeDtypeStruct((n, d), jnp.float32),
        scratch_types=[
            pltpu.VMEM((per_w,), jnp.int32),
            pltpu.VMEM((_CPW, _CHUNK, d), jnp.float32),
            pltpu.SemaphoreType.DMA,
            pltpu.SemaphoreType.DMA,
        ],
    )
    def gk(table_hbm, idx_hbm, out_hbm, idx_v, rows_v, gsem, wsem):
        wid = lax.axis_index("s") * 2 + lax.axis_index("c")
        base = wid * per_w

        @pl.when(wid < tail_wid)
        def _full():
            pltpu.sync_copy(idx_hbm.at[pl.ds(base, per_w)], idx_v)
            gathers = [
                pltpu.async_copy(
                    table_hbm.at[idx_v.at[pl.ds(c * _CHUNK, _CHUNK)]],
                    rows_v.at[c], gsem)
                for c in range(_CPW)
            ]
            writes = []
            for c in range(_CPW):
                gathers[c].wait()
                writes.append(pltpu.async_copy(
                    rows_v.at[c],
                    out_hbm.at[pl.ds(base + c * _CHUNK, _CHUNK)],
                    wsem))
            for wr in writes:
                wr.wait()

        @pl.when(wid == tail_wid)
        def _tail():
            for c in range(tail_count):
                pltpu.sync_copy(
                    idx_hbm.at[pl.ds(base + c * _CHUNK, _CHUNK)],
                    idx_v.at[pl.ds(c * _CHUNK, _CHUNK)])
                pltpu.async_copy(
                    table_hbm.at[idx_v.at[pl.ds(c * _CHUNK, _CHUNK)]],
                    rows_v.at[c], gsem).wait()
                pltpu.sync_copy(
                    rows_v.at[c],
                    out_hbm.at[pl.ds(base + c * _CHUNK, _CHUNK)])

    return gk


def kernel(x, selected_index, support, w):
    n = x.shape[0]
    s = _spmm(support[0], x, w)
    for i in range(1, support.shape[0]):
        s = s + _spmm(support[i], x, w)
    return _make_gather(n, w.shape[1])(s, selected_index.astype(jnp.int32))
